# baseline (speedup 1.0000x reference)
import functools

import jax
import jax.numpy as jnp
from jax import lax
from jax.experimental import pallas as pl
from jax.experimental.pallas import tpu as pltpu
from jax.experimental.pallas import tpu_sc as plsc

HID = 768
TOK = 4 * 2048

NC = 2
NS = 16
NW = NC * NS
TOK_PER_W = TOK // NW
CHUNK = 32
NCHUNK = TOK_PER_W // CHUNK
LANES = 16
VPT = HID // LANES
NBW = 3


def _sc_body(ids_hbm, pids_hbm, tids_hbm, wtab, ptab, ttab, out_hbm,
             idx_w, idx_p, tf_vmem, type_v,
             bufw0, bufw1, bufw2, bufp0, bufp1,
             semw0, semw1, semw2, semp0, semp1,
             semo0, semo1, semo2, semst):
    wid = lax.axis_index("s") * NC + lax.axis_index("c")
    row0 = wid * NCHUNK
    base = wid * TOK_PER_W

    bufw = (bufw0, bufw1, bufw2)
    bufp = (bufp0, bufp1)
    semw = (semw0, semw1, semw2)
    semp = (semp0, semp1)
    semo = (semo0, semo1, semo2)

    st1 = pltpu.async_copy(ids_hbm.at[pl.ds(row0, NCHUNK)], idx_w, semst)
    st2 = pltpu.async_copy(pids_hbm.at[pl.ds(row0, NCHUNK)], idx_p, semst)
    st3 = pltpu.async_copy(tids_hbm.at[pl.ds(base, TOK_PER_W)], tf_vmem, semst)
    st4 = pltpu.async_copy(ttab, type_v, semst)

    def start_word(c):
        return pltpu.async_copy(wtab.at[idx_w.at[c]], bufw[c % NBW],
                                semw[c % NBW])

    def start_pos(c):
        return pltpu.async_copy(ptab.at[idx_p.at[c]], bufp[c % 2],
                                semp[c % 2])

    def compute(c):
        bw, bp = c % NBW, c % 2

        def add_k(k, carry):
            sl = pl.ds(k * LANES, LANES)
            t0k = type_v[0, sl]
            dk = type_v[1, sl] - t0k
            for g in range(CHUNK // LANES):
                tfv = tf_vmem[pl.ds(c * CHUNK + g * LANES, LANES)].astype(
                    jnp.float32)
                for l in range(LANES):
                    j = g * LANES + l
                    tf = jnp.full((LANES,), tfv[l], jnp.float32)
                    bufw[bw][j, sl] = (bufw[bw][j, sl] + bufp[bp][j, sl]
                                       + (t0k + tf * dk))
            return carry

        lax.fori_loop(0, VPT, add_k, 0)

    st1.wait()
    st2.wait()
    W = {c: start_word(c) for c in range(min(NBW, NCHUNK))}
    P = {c: start_pos(c) for c in range(min(2, NCHUNK))}
    st3.wait()
    st4.wait()
    O = {}
    for c in range(NCHUNK):
        W.pop(c).wait()
        P.pop(c).wait()
        compute(c)
        O[c] = pltpu.async_copy(
            bufw[c % NBW], out_hbm.at[pl.ds(base + c * CHUNK, CHUNK)],
            semo[c % NBW])
        if c + 2 < NCHUNK:
            P[c + 2] = start_pos(c + 2)
        if c >= 1 and c + 2 < NCHUNK and c + 2 not in W:
            O[c - 1].wait()
            W[c + 2] = start_word(c + 2)
    for c in range(max(NCHUNK - 3, 0), NCHUNK):
        O[c].wait()


def kernel(input_ids, position_ids, token_type_ids, word_embeddings,
           token_type_embeddings, position_embeddings):
    b, s = input_ids.shape
    ids = input_ids.reshape(TOK // CHUNK, CHUNK).astype(jnp.int32)
    pids = position_ids.reshape(TOK // CHUNK, CHUNK).astype(jnp.int32)
    tids = token_type_ids.reshape(TOK).astype(jnp.int32)

    mesh = plsc.VectorSubcoreMesh(core_axis_name="c", subcore_axis_name="s")
    run = functools.partial(
        pl.kernel,
        mesh=mesh,
        out_type=jax.ShapeDtypeStruct((TOK, HID), jnp.float32),
        compiler_params=pltpu.CompilerParams(needs_layout_passes=False),
        scratch_types=[
            pltpu.VMEM((NCHUNK, CHUNK), jnp.int32),
            pltpu.VMEM((NCHUNK, CHUNK), jnp.int32),
            pltpu.VMEM((TOK_PER_W,), jnp.int32),
            pltpu.VMEM((2, HID), jnp.float32),
            pltpu.VMEM((CHUNK, HID), jnp.float32),
            pltpu.VMEM((CHUNK, HID), jnp.float32),
            pltpu.VMEM((CHUNK, HID), jnp.float32),
            pltpu.VMEM((CHUNK, HID), jnp.float32),
            pltpu.VMEM((CHUNK, HID), jnp.float32),
            pltpu.SemaphoreType.DMA,
            pltpu.SemaphoreType.DMA,
            pltpu.SemaphoreType.DMA,
            pltpu.SemaphoreType.DMA,
            pltpu.SemaphoreType.DMA,
            pltpu.SemaphoreType.DMA,
            pltpu.SemaphoreType.DMA,
            pltpu.SemaphoreType.DMA,
            pltpu.SemaphoreType.DMA,
        ],
    )(_sc_body)

    out = run(ids, pids, tids,
              word_embeddings.astype(jnp.float32),
              position_embeddings.astype(jnp.float32),
              token_type_embeddings.astype(jnp.float32))
    return out.reshape(b, s, HID)
